# baseline (speedup 1.0000x reference)
import functools

import jax
import jax.numpy as jnp
from jax import lax
from jax.experimental import pallas as pl
from jax.experimental.pallas import tpu as pltpu
from jax.experimental.pallas import tpu_sc as plsc

D = 128
NB = 4
NS = 16
NC = 2
CHUNK = 32
SUPER = 8
SPAN = 632
NPAD = NS * SPAN
CROWS = 80
NRELP = 480


def _mm_body(x_ref, w_ref, o_ref):
    o_ref[...] = jnp.dot(x_ref[...], w_ref[...],
                         preferred_element_type=jnp.float32)


def _matmul(x, w, block=400):
    m, k = x.shape
    _, n = w.shape
    return pl.pallas_call(
        _mm_body,
        grid=(m // block,),
        in_specs=[pl.BlockSpec((block, k), lambda i: (i, 0)),
                  pl.BlockSpec((k, n), lambda i: (0, 0))],
        out_specs=pl.BlockSpec((block, n), lambda i: (i, 0)),
        out_shape=jax.ShapeDtypeStruct((m, n), jnp.float32),
    )(x, w)


def _upd_body(s_ref, cnt_ref, x_ref, root_ref, bias_ref, o_ref, *, relu):
    cnt = jnp.maximum(cnt_ref[...], 1.0)
    aggr = s_ref[...] / cnt
    o = aggr + jnp.dot(x_ref[...], root_ref[...],
                       preferred_element_type=jnp.float32) + bias_ref[...]
    if relu:
        o = jnp.maximum(o, 0.0)
    o_ref[...] = o


def _update(s, cnt, x, root, bias, relu, block=400):
    m = x.shape[0]
    n = cnt.shape[0]
    nb = n // block
    return pl.pallas_call(
        functools.partial(_upd_body, relu=relu),
        grid=(m // block,),
        in_specs=[pl.BlockSpec((block, D), lambda i: (i, 0)),
                  pl.BlockSpec((block, 1), lambda i: (i % nb, 0)),
                  pl.BlockSpec((block, D), lambda i: (i, 0)),
                  pl.BlockSpec((D, D), lambda i: (0, 0)),
                  pl.BlockSpec((1, D), lambda i: (0, 0))],
        out_specs=pl.BlockSpec((block, D), lambda i: (i, 0)),
        out_shape=jax.ShapeDtypeStruct((m, D), jnp.float32),
    )(s, cnt, x, root, bias.reshape(1, D))


def _sc_edge_body(ya, yb, src2, dst2, etype2, norm2, att, zeros, out, out_cnt,
                  ssrc, sdst, setp, snrm, attbuf, rowsa, rowsb, msg,
                  cntbuf, idxbuf, acc, acc2, sem, semb):
    cid = lax.axis_index("c")
    tid = lax.axis_index("s")
    n = ya.shape[0] // 2
    rows_per_tile = src2.shape[0] // NS
    nsup = rows_per_tile // SUPER

    pltpu.sync_copy(att, attbuf)

    pltpu.sync_copy(zeros.at[pl.ds(tid * SPAN, SPAN)],
                    acc.at[pl.ds(tid * SPAN, SPAN)])
    pltpu.sync_copy(zeros.at[pl.ds(0, CROWS)], cntbuf)

    @pl.when(tid == 0)
    def _():
        pltpu.sync_copy(zeros.at[pl.ds(0, CROWS)], acc2)

    iota16 = lax.iota(jnp.int32, 16)
    for j in range(CROWS // 16):
        idxbuf[pl.ds(j * 16, 16)] = iota16 + j * 16

    plsc.subcore_barrier()

    is_imag = cid == 1
    off = cid * n
    ones16 = jnp.ones((16,), jnp.float32)
    lane0 = iota16 == 0
    flagv = attbuf[pl.ds((NRELP - 2) * 8, 16)]
    do_cnt = (cid == 0) & (flagv[0] > 0.5)

    def _super(i, c):
        row0 = tid * rows_per_tile + i * SUPER
        pltpu.sync_copy(src2.at[pl.ds(row0, SUPER)], ssrc)
        pltpu.sync_copy(dst2.at[pl.ds(row0, SUPER)], sdst)
        pltpu.sync_copy(etype2.at[pl.ds(row0, SUPER)], setp)
        pltpu.sync_copy(norm2.at[pl.ds(row0, SUPER)], snrm)
        for j in range(SUPER):
            for g in range(CHUNK // 16):
                ssrc[j, pl.ds(g * 16, 16)] = (
                    ssrc[j, pl.ds(g * 16, 16)] + off)

        def _chunk(j, cc2):
            pltpu.async_copy(ya.at[ssrc.at[j]], rowsa, sem).wait()
            pltpu.async_copy(yb.at[ssrc.at[j]], rowsb, semb).wait()

            @pl.when(do_cnt)
            def _():
                for q in range(CHUNK // 16):
                    dv = sdst[j, pl.ds(q * 16, 16)]
                    for kk in range(16):
                        dvk = dv[kk]
                        i0v = jnp.full((16,), dvk >> 7, jnp.int32)
                        i1v = jnp.full((16,), dvk & 127, jnp.int32)
                        plsc.addupdate_scatter(cntbuf, [i0v, i1v], ones16,
                                               mask=lane0)

            for q in range(CHUNK // 16):
                etv = setp[j, pl.ds(q * 16, 16)]
                nv = snrm[j, pl.ds(q * 16, 16)]
                sv = jnp.where(is_imag, nv, ones16)
                tbase = etv * 8
                cv0 = plsc.load_gather(attbuf, [tbase]) * sv
                cv1 = plsc.load_gather(attbuf, [tbase + 1]) * sv
                cv2 = plsc.load_gather(attbuf, [tbase + 2]) * sv
                cv3 = plsc.load_gather(attbuf, [tbase + 3]) * sv
                kvec = iota16 + q * 16

                def _col(c4, cc3):
                    for u in range(4):
                        cf = jnp.full((16,), c4 * 4 + u, jnp.int32)
                        va = plsc.load_gather(rowsa, [kvec, cf])
                        vb = plsc.load_gather(rowsa, [kvec, cf + D])
                        vc = plsc.load_gather(rowsb, [kvec, cf])
                        vd = plsc.load_gather(rowsb, [kvec, cf + D])
                        m = va * cv0 + vb * cv1 + vc * cv2 + vd * cv3
                        plsc.store_scatter(msg, [kvec, cf], m)
                    return cc3
                lax.fori_loop(0, D // 4, _col, 0)

            pltpu.sync_copy(msg, acc.at[sdst.at[j]], add=True)
            return cc2
        lax.fori_loop(0, SUPER, _chunk, 0)
        return c
    lax.fori_loop(0, nsup, _super, 0)

    pltpu.sync_copy(cntbuf, acc2.at[idxbuf], add=True)
    plsc.subcore_barrier()

    tail = n - (NS - 1) * SPAN

    @pl.when(tid < NS - 1)
    def _():
        pltpu.sync_copy(acc.at[pl.ds(tid * SPAN, SPAN)],
                        out.at[pl.ds(cid * n + tid * SPAN, SPAN)])

    @pl.when(tid == NS - 1)
    def _():
        pltpu.sync_copy(acc.at[pl.ds((NS - 1) * SPAN, tail)],
                        out.at[pl.ds(cid * n + (NS - 1) * SPAN, tail)])

    @pl.when((cid == 0) & (tid < CROWS // 8))
    def _():
        pltpu.sync_copy(acc2.at[pl.ds(tid * 8, 8)],
                        out_cnt.at[pl.ds(tid * 8, 8)])


@functools.lru_cache(maxsize=None)
def _make_sc_edge(n, e):
    mesh = plsc.VectorSubcoreMesh(core_axis_name="c", subcore_axis_name="s",
                                  num_cores=NC, num_subcores=NS)
    return pl.kernel(
        _sc_edge_body,
        out_type=(jax.ShapeDtypeStruct((2 * n, D), jnp.float32),
                  jax.ShapeDtypeStruct((CROWS, D), jnp.float32)),
        mesh=mesh,
        compiler_params=pltpu.CompilerParams(needs_layout_passes=False),
        scratch_types=[
            pltpu.VMEM((SUPER, CHUNK), jnp.int32),
            pltpu.VMEM((SUPER, CHUNK), jnp.int32),
            pltpu.VMEM((SUPER, CHUNK), jnp.int32),
            pltpu.VMEM((SUPER, CHUNK), jnp.float32),
            pltpu.VMEM((NRELP * 8,), jnp.float32),
            pltpu.VMEM((CHUNK, NB * D // 2), jnp.float32),
            pltpu.VMEM((CHUNK, NB * D // 2), jnp.float32),
            pltpu.VMEM((CHUNK, D), jnp.float32),
            pltpu.VMEM((CROWS, D), jnp.float32),
            pltpu.VMEM((CROWS,), jnp.int32),
            pltpu.VMEM_SHARED((NPAD, D), jnp.float32),
            pltpu.VMEM_SHARED((CROWS, D), jnp.float32),
            pltpu.SemaphoreType.DMA,
            pltpu.SemaphoreType.DMA,
        ],
    )


def kernel(entity, edge_index, edge_type, edge_norm, emb_e_real, emb_e_img,
           basis1, att1, root1, bias1, basis2, att2, root2, bias2):
    xr = jnp.take(emb_e_real, entity, axis=0)
    xi = jnp.take(emb_e_img, entity, axis=0)
    x = jnp.concatenate([xr, xi], axis=0)
    n = xr.shape[0]
    e = edge_type.shape[0]
    grp = NS * CHUNK * SUPER
    epad = -e % grp
    ip = jnp.full((epad,), 0, jnp.int32)
    src2 = jnp.concatenate([edge_index[0], ip]).reshape(-1, CHUNK)
    dst2 = jnp.concatenate(
        [edge_index[1], jnp.full((epad,), n, jnp.int32)]).reshape(-1, CHUNK)
    etype2 = jnp.concatenate([edge_type, ip]).reshape(-1, CHUNK)
    norm2 = jnp.concatenate(
        [edge_norm, jnp.zeros((epad,), jnp.float32)]).reshape(-1, CHUNK)
    e = e + epad

    bcat1 = basis1.transpose(1, 0, 2).reshape(D, NB * D)
    bcat2 = basis2.transpose(1, 0, 2).reshape(D, NB * D)
    attf1 = jnp.pad(att1, ((0, NRELP - att1.shape[0]), (0, 8 - NB)))
    attf2 = jnp.pad(att2, ((0, NRELP - att2.shape[0]), (0, 8 - NB)))
    attf1 = attf1.reshape(-1)
    attf2 = attf2.reshape(-1)
    attf1 = attf1.at[(NRELP - 2) * 8].set(1.0)

    sc_edge = _make_sc_edge(n, e)
    zeros = jnp.zeros((NPAD, D), jnp.float32)
    half = NB * D // 2

    y1a = _matmul(x, bcat1[:, :half])
    y1b = _matmul(x, bcat1[:, half:])
    s1, cnt_grid = sc_edge(y1a, y1b, src2, dst2, etype2, norm2, attf1, zeros)
    cnt = cnt_grid.reshape(-1)[:n, None]
    x1 = _update(s1, cnt, x, root1, bias1, relu=True)

    y2a = _matmul(x1, bcat2[:, :half])
    y2b = _matmul(x1, bcat2[:, half:])
    s2, _ = sc_edge(y2a, y2b, src2, dst2, etype2, norm2, attf2, zeros)
    x2 = _update(s2, cnt, x1, root2, bias2, relu=False)

    return x2[:n], x2[n:]


# speedup vs baseline: 2.6695x; 2.6695x over previous
import functools

import jax
import jax.numpy as jnp
from jax import lax
from jax.experimental import pallas as pl
from jax.experimental.pallas import tpu as pltpu
from jax.experimental.pallas import tpu_sc as plsc

D = 128
NB = 4
NS = 16
NC = 2
CHUNK = 32
SUPER = 8
SPAN = 632
NPAD = NS * SPAN
CROWS = 80
NRELP = 480


def _mm_body(x_ref, w_ref, o_ref):
    o_ref[...] = jnp.dot(x_ref[...], w_ref[...],
                         preferred_element_type=jnp.float32)


def _matmul(x, w, block=400):
    m, k = x.shape
    _, n = w.shape
    return pl.pallas_call(
        _mm_body,
        grid=(m // block,),
        in_specs=[pl.BlockSpec((block, k), lambda i: (i, 0)),
                  pl.BlockSpec((k, n), lambda i: (0, 0))],
        out_specs=pl.BlockSpec((block, n), lambda i: (i, 0)),
        out_shape=jax.ShapeDtypeStruct((m, n), jnp.float32),
    )(x, w)


def _upd_body(s_ref, cnt_ref, x_ref, root_ref, bias_ref, o_ref, *, relu):
    cnt = jnp.maximum(cnt_ref[...], 1.0)
    aggr = s_ref[...] / cnt
    o = aggr + jnp.dot(x_ref[...], root_ref[...],
                       preferred_element_type=jnp.float32) + bias_ref[...]
    if relu:
        o = jnp.maximum(o, 0.0)
    o_ref[...] = o


def _update(s, cnt, x, root, bias, relu, block=400):
    m = x.shape[0]
    n = cnt.shape[0]
    nb = n // block
    return pl.pallas_call(
        functools.partial(_upd_body, relu=relu),
        grid=(m // block,),
        in_specs=[pl.BlockSpec((block, D), lambda i: (i, 0)),
                  pl.BlockSpec((block, 1), lambda i: (i % nb, 0)),
                  pl.BlockSpec((block, D), lambda i: (i, 0)),
                  pl.BlockSpec((D, D), lambda i: (0, 0)),
                  pl.BlockSpec((1, D), lambda i: (0, 0))],
        out_specs=pl.BlockSpec((block, D), lambda i: (i, 0)),
        out_shape=jax.ShapeDtypeStruct((m, D), jnp.float32),
    )(s, cnt, x, root, bias.reshape(1, D))


def _sc_edge_body(y, src2, dst2, etype2, norm2, att, zeros, out,
                  ssrc, sdst, setp, snrm, attbuf, coefb, rows0, rows1, msg,
                  acc, sem0, sem1):
    cid = lax.axis_index("c")
    tid = lax.axis_index("s")
    n = y.shape[0] // 2
    rows_per_tile = src2.shape[0] // NS
    nsup = rows_per_tile // SUPER

    pltpu.sync_copy(att, attbuf)

    pltpu.sync_copy(zeros.at[pl.ds(tid * SPAN, SPAN)],
                    acc.at[pl.ds(tid * SPAN, SPAN)])

    plsc.subcore_barrier()

    is_imag = cid == 1
    off = cid * n
    ones16 = jnp.ones((16,), jnp.float32)
    iota16 = lax.iota(jnp.int32, 16)

    def _combine(j, rows):
        def _coef(q, cc):
            etv = setp[j, pl.ds(q * 16, 16)]
            nv = snrm[j, pl.ds(q * 16, 16)]
            sv = jnp.where(is_imag, nv, ones16)
            tbase = etv * 8
            kbase = (iota16 + q * 16) * 8
            for b in range(NB):
                cb = plsc.load_gather(attbuf, [tbase + b]) * sv
                plsc.store_scatter(coefb, [kbase + b], cb)
            return cc
        lax.fori_loop(0, CHUNK // 16, _coef, 0)

        def _grp(q, cc):
            for kk in range(16):
                k = q * 16 + kk
                kf = jnp.full((16,), k * 8, jnp.int32)
                c0 = plsc.load_gather(coefb, [kf])
                c1 = plsc.load_gather(coefb, [kf + 1])
                c2 = plsc.load_gather(coefb, [kf + 2])
                c3 = plsc.load_gather(coefb, [kf + 3])
                for jj in range(D // 16):
                    v = rows[k, pl.ds(jj * 16, 16)] * c0
                    v = v + rows[k, pl.ds(D + jj * 16, 16)] * c1
                    v = v + rows[k, pl.ds(2 * D + jj * 16, 16)] * c2
                    v = v + rows[k, pl.ds(3 * D + jj * 16, 16)] * c3
                    msg[k, pl.ds(jj * 16, 16)] = v
            return cc
        lax.fori_loop(0, CHUNK // 16, _grp, 0)

        pltpu.sync_copy(msg, acc.at[sdst.at[j]], add=True)

    def _super(i, c):
        row0 = tid * rows_per_tile + i * SUPER
        pltpu.sync_copy(src2.at[pl.ds(row0, SUPER)], ssrc)
        pltpu.sync_copy(dst2.at[pl.ds(row0, SUPER)], sdst)
        pltpu.sync_copy(etype2.at[pl.ds(row0, SUPER)], setp)
        pltpu.sync_copy(norm2.at[pl.ds(row0, SUPER)], snrm)
        for j in range(SUPER):
            for g in range(CHUNK // 16):
                ssrc[j, pl.ds(g * 16, 16)] = (
                    ssrc[j, pl.ds(g * 16, 16)] + off)

        pltpu.async_copy(y.at[ssrc.at[0]], rows0, sem0)

        def _pair(p, cc):
            j0 = 2 * p
            j1 = j0 + 1
            pltpu.async_copy(y.at[ssrc.at[j1]], rows1, sem1)
            pltpu.make_async_copy(y.at[ssrc.at[j0]], rows0, sem0).wait()
            _combine(j0, rows0)

            @pl.when(p < SUPER // 2 - 1)
            def _():
                pltpu.async_copy(y.at[ssrc.at[j0 + 2]], rows0, sem0)

            pltpu.make_async_copy(y.at[ssrc.at[j1]], rows1, sem1).wait()
            _combine(j1, rows1)
            return cc
        lax.fori_loop(0, SUPER // 2, _pair, 0)
        return c
    lax.fori_loop(0, nsup, _super, 0)

    plsc.subcore_barrier()

    tail = n - (NS - 1) * SPAN

    @pl.when(tid < NS - 1)
    def _():
        pltpu.sync_copy(acc.at[pl.ds(tid * SPAN, SPAN)],
                        out.at[pl.ds(cid * n + tid * SPAN, SPAN)])

    @pl.when(tid == NS - 1)
    def _():
        pltpu.sync_copy(acc.at[pl.ds((NS - 1) * SPAN, tail)],
                        out.at[pl.ds(cid * n + (NS - 1) * SPAN, tail)])


@functools.lru_cache(maxsize=None)
def _make_sc_edge(n, e):
    mesh = plsc.VectorSubcoreMesh(core_axis_name="c", subcore_axis_name="s",
                                  num_cores=NC, num_subcores=NS)
    return pl.kernel(
        _sc_edge_body,
        out_type=jax.ShapeDtypeStruct((2 * n, D), jnp.float32),
        mesh=mesh,
        compiler_params=pltpu.CompilerParams(needs_layout_passes=False),
        scratch_types=[
            pltpu.VMEM((SUPER, CHUNK), jnp.int32),
            pltpu.VMEM((SUPER, CHUNK), jnp.int32),
            pltpu.VMEM((SUPER, CHUNK), jnp.int32),
            pltpu.VMEM((SUPER, CHUNK), jnp.float32),
            pltpu.VMEM((NRELP * 8,), jnp.float32),
            pltpu.VMEM((CHUNK * 8,), jnp.float32),
            pltpu.VMEM((CHUNK, NB * D), jnp.float32),
            pltpu.VMEM((CHUNK, NB * D), jnp.float32),
            pltpu.VMEM((CHUNK, D), jnp.float32),
            pltpu.VMEM_SHARED((NPAD, D), jnp.float32),
            pltpu.SemaphoreType.DMA,
            pltpu.SemaphoreType.DMA,
        ],
    )


def _sc_cnt_body(dst2, zeros, out_cnt, sdstc, cntbuf, idxbuf, acc2, sem):
    cid = lax.axis_index("c")
    tid = lax.axis_index("s")
    rows_per_tile = dst2.shape[0] // NS
    nsup = rows_per_tile // SUPER

    pltpu.sync_copy(zeros.at[pl.ds(0, CROWS)], cntbuf)

    @pl.when(tid == 0)
    def _():
        pltpu.sync_copy(zeros.at[pl.ds(0, CROWS)], acc2)

    iota16 = lax.iota(jnp.int32, 16)
    for j in range(CROWS // 16):
        idxbuf[pl.ds(j * 16, 16)] = iota16 + j * 16

    plsc.subcore_barrier()

    ones16 = jnp.ones((16,), jnp.float32)
    lane0 = iota16 == 0

    @pl.when(cid == 0)
    def _():
        def _sup(i, c):
            row0 = tid * rows_per_tile + i * SUPER
            pltpu.sync_copy(dst2.at[pl.ds(row0, SUPER)], sdstc)
            for j in range(SUPER):
                for q in range(CHUNK // 16):
                    dv = sdstc[j, pl.ds(q * 16, 16)]
                    for kk in range(16):
                        dvk = dv[kk]
                        i0v = jnp.full((16,), dvk >> 7, jnp.int32)
                        i1v = jnp.full((16,), dvk & 127, jnp.int32)
                        plsc.addupdate_scatter(cntbuf, [i0v, i1v], ones16,
                                               mask=lane0)
            return c
        lax.fori_loop(0, nsup, _sup, 0)
        pltpu.sync_copy(cntbuf, acc2.at[idxbuf], add=True)

    plsc.subcore_barrier()

    @pl.when((cid == 0) & (tid < CROWS // 8))
    def _():
        pltpu.sync_copy(acc2.at[pl.ds(tid * 8, 8)],
                        out_cnt.at[pl.ds(tid * 8, 8)])


@functools.lru_cache(maxsize=None)
def _make_sc_cnt(e):
    mesh = plsc.VectorSubcoreMesh(core_axis_name="c", subcore_axis_name="s",
                                  num_cores=NC, num_subcores=NS)
    return pl.kernel(
        _sc_cnt_body,
        out_type=jax.ShapeDtypeStruct((CROWS, D), jnp.float32),
        mesh=mesh,
        compiler_params=pltpu.CompilerParams(needs_layout_passes=False),
        scratch_types=[
            pltpu.VMEM((SUPER, CHUNK), jnp.int32),
            pltpu.VMEM((CROWS, D), jnp.float32),
            pltpu.VMEM((CROWS,), jnp.int32),
            pltpu.VMEM_SHARED((CROWS, D), jnp.float32),
            pltpu.SemaphoreType.DMA,
        ],
    )


def kernel(entity, edge_index, edge_type, edge_norm, emb_e_real, emb_e_img,
           basis1, att1, root1, bias1, basis2, att2, root2, bias2):
    xr = jnp.take(emb_e_real, entity, axis=0)
    xi = jnp.take(emb_e_img, entity, axis=0)
    x = jnp.concatenate([xr, xi], axis=0)
    n = xr.shape[0]
    e = edge_type.shape[0]
    grp = NS * CHUNK * SUPER
    epad = -e % grp
    ip = jnp.full((epad,), 0, jnp.int32)
    src2 = jnp.concatenate([edge_index[0], ip]).reshape(-1, CHUNK)
    dst2 = jnp.concatenate(
        [edge_index[1], jnp.full((epad,), n, jnp.int32)]).reshape(-1, CHUNK)
    etype2 = jnp.concatenate([edge_type, ip]).reshape(-1, CHUNK)
    norm2 = jnp.concatenate(
        [edge_norm, jnp.zeros((epad,), jnp.float32)]).reshape(-1, CHUNK)
    e = e + epad

    bcat1 = basis1.transpose(1, 0, 2).reshape(D, NB * D)
    bcat2 = basis2.transpose(1, 0, 2).reshape(D, NB * D)
    attf1 = jnp.pad(att1, ((0, NRELP - att1.shape[0]), (0, 8 - NB)))
    attf2 = jnp.pad(att2, ((0, NRELP - att2.shape[0]), (0, 8 - NB)))
    attf1 = attf1.reshape(-1)
    attf2 = attf2.reshape(-1)

    sc_edge = _make_sc_edge(n, e)
    sc_cnt = _make_sc_cnt(e)
    zeros = jnp.zeros((NPAD, D), jnp.float32)

    cnt_grid = sc_cnt(dst2, zeros)
    cnt = cnt_grid.reshape(-1)[:n, None]

    y1 = _matmul(x, bcat1)
    s1 = sc_edge(y1, src2, dst2, etype2, norm2, attf1, zeros)
    x1 = _update(s1, cnt, x, root1, bias1, relu=True)

    y2 = _matmul(x1, bcat2)
    s2 = sc_edge(y2, src2, dst2, etype2, norm2, attf2, zeros)
    x2 = _update(s2, cnt, x1, root2, bias2, relu=False)

    return x2[:n], x2[n:]
